# Initial kernel scaffold; baseline (speedup 1.0000x reference)
import jax
import jax.numpy as jnp
from jax.experimental import pallas as pl


def kernel(X, lut):
    raise NotImplementedError("write your pallas kernel here")


# speedup vs baseline: 1.1847x; 1.1847x over previous
import functools
import math

import jax
import jax.numpy as jnp
from jax import lax
from jax.experimental import pallas as pl
from jax.experimental.pallas import tpu as pltpu
from jax.experimental.pallas import tpu_sc as plsc

_info = plsc.get_sparse_core_info()
_NC, _NS, _L = _info.num_cores, _info.num_subcores, _info.num_lanes
_NW = _NC * _NS


def _make_emb_kernel(B, V, D, b_per_w, C):
    nch = b_per_w // C
    scale = math.sqrt(D)
    mesh = plsc.VectorSubcoreMesh(core_axis_name="c", subcore_axis_name="s")

    @functools.partial(
        pl.kernel,
        out_type=jax.ShapeDtypeStruct((B, D), jnp.float32),
        mesh=mesh,
        scratch_types=[
            pltpu.VMEM((nch, C), jnp.int32),
            pltpu.VMEM((C, D), jnp.float32),
            pltpu.SemaphoreType.DMA,
        ],
    )
    def emb(idx_hbm, lut_hbm, out_hbm, idx_v, rows_v, sem):
        wid = lax.axis_index("s") * _NC + lax.axis_index("c")
        base = wid * b_per_w
        pltpu.sync_copy(idx_hbm.at[wid], idx_v)

        def chunk_body(i, _):
            pltpu.async_copy(lut_hbm.at[idx_v.at[i]], rows_v, sem).wait()

            def row_body(r, _):
                for j in range(D // _L):
                    sl = pl.ds(j * _L, _L)
                    rows_v[r, sl] = rows_v[r, sl] * scale
                return _

            lax.fori_loop(0, C, row_body, 0, unroll=False)

            pltpu.sync_copy(rows_v, out_hbm.at[pl.ds(base + i * C, C)])
            return _

        lax.fori_loop(0, nch, chunk_body, 0, unroll=False)

    return emb


@jax.jit
def kernel(X, lut):
    S0, S1 = X.shape
    V, D = lut.shape
    B = S0 * S1
    b_per_w = B // _NW
    C = 64
    idx = X.reshape(_NW, b_per_w // C, C).astype(jnp.int32)
    emb = _make_emb_kernel(B, V, D, b_per_w, C)
    out = emb(idx, lut)
    return out.reshape(S0, S1, D)


# speedup vs baseline: 1.3087x; 1.1047x over previous
import functools
import math

import jax
import jax.numpy as jnp
from jax import lax
from jax.experimental import pallas as pl
from jax.experimental.pallas import tpu as pltpu
from jax.experimental.pallas import tpu_sc as plsc

_info = plsc.get_sparse_core_info()
_NC, _NS, _L = _info.num_cores, _info.num_subcores, _info.num_lanes
_NW = _NC * _NS


def _make_emb_kernel(B, V, D, b_per_w, C, NBUF):
    nch = b_per_w // C
    scale = math.sqrt(D)
    mesh = plsc.VectorSubcoreMesh(core_axis_name="c", subcore_axis_name="s")

    @functools.partial(
        pl.kernel,
        out_type=jax.ShapeDtypeStruct((B, D), jnp.float32),
        mesh=mesh,
        scratch_types=[
            pltpu.VMEM((nch, C), jnp.int32),
        ]
        + [pltpu.VMEM((C, D), jnp.float32) for _ in range(NBUF)]
        + [pltpu.SemaphoreType.DMA for _ in range(2 * NBUF)],
    )
    def emb(idx_hbm, lut_hbm, out_hbm, idx_v, *scratch):
        bufs = scratch[:NBUF]
        gsems = scratch[NBUF : 2 * NBUF]
        ssems = scratch[2 * NBUF :]
        wid = lax.axis_index("s") * _NC + lax.axis_index("c")
        base = wid * b_per_w
        pltpu.sync_copy(idx_hbm.at[wid], idx_v)

        def scale_buf(buf):
            def row_body(r, _):
                for j in range(D // _L):
                    sl = pl.ds(j * _L, _L)
                    buf[r, sl] = buf[r, sl] * scale
                return _

            lax.fori_loop(0, C, row_body, 0, unroll=False)

        gathers = [None] * nch
        scatters = [None] * nch
        for c in range(min(NBUF, nch)):
            gathers[c] = pltpu.async_copy(
                lut_hbm.at[idx_v.at[c]], bufs[c % NBUF], gsems[c % NBUF]
            )
        for c in range(nch):
            b = c % NBUF
            gathers[c].wait()
            scale_buf(bufs[b])
            scatters[c] = pltpu.async_copy(
                bufs[b], out_hbm.at[pl.ds(base + c * C, C)], ssems[b]
            )
            d = c + 1
            if d < nch and d >= NBUF:
                bd = d % NBUF
                scatters[d - NBUF].wait()
                gathers[d] = pltpu.async_copy(
                    lut_hbm.at[idx_v.at[d]], bufs[bd], gsems[bd]
                )
        for c in range(max(0, nch - NBUF), nch):
            scatters[c].wait()

    return emb


@jax.jit
def kernel(X, lut):
    S0, S1 = X.shape
    V, D = lut.shape
    B = S0 * S1
    b_per_w = B // _NW
    C = 64
    NBUF = 3
    idx = X.reshape(_NW, b_per_w // C, C).astype(jnp.int32)
    emb = _make_emb_kernel(B, V, D, b_per_w, C, NBUF)
    out = emb(idx, lut)
    return out.reshape(S0, S1, D)


# speedup vs baseline: 1.4823x; 1.1327x over previous
import functools
import math

import jax
import jax.numpy as jnp
from jax import lax
from jax.experimental import pallas as pl
from jax.experimental.pallas import tpu as pltpu
from jax.experimental.pallas import tpu_sc as plsc

_info = plsc.get_sparse_core_info()
_NC, _NS, _L = _info.num_cores, _info.num_subcores, _info.num_lanes
_NW = _NC * _NS


def _make_emb_kernel(B, V, D, b_per_w, C, NBUF):
    nch = b_per_w // C
    scale = math.sqrt(D)
    mesh = plsc.VectorSubcoreMesh(core_axis_name="c", subcore_axis_name="s")

    @functools.partial(
        pl.kernel,
        out_type=jax.ShapeDtypeStruct((B, D), jnp.float32),
        mesh=mesh,
        scratch_types=[
            pltpu.VMEM((nch, C), jnp.int32),
        ]
        + [pltpu.VMEM((C, D), jnp.float32) for _ in range(NBUF)]
        + [pltpu.SemaphoreType.DMA for _ in range(2 * NBUF)],
    )
    def emb(idx_hbm, lut_hbm, out_hbm, idx_v, *scratch):
        bufs = scratch[:NBUF]
        gsems = scratch[NBUF : 2 * NBUF]
        ssems = scratch[2 * NBUF :]
        wid = lax.axis_index("s") * _NC + lax.axis_index("c")
        base = wid * b_per_w
        pltpu.sync_copy(idx_hbm.at[wid], idx_v)

        def scale_buf(buf):
            def row_body(r, _):
                for j in range(D // _L):
                    sl = pl.ds(j * _L, _L)
                    buf[r, sl] = buf[r, sl] * scale
                return _

            lax.fori_loop(0, C, row_body, 0, unroll=False)

        gathers = [None] * nch
        scatters = [None] * nch
        for c in range(min(NBUF - 1, nch)):
            gathers[c] = pltpu.async_copy(
                lut_hbm.at[idx_v.at[c]], bufs[c % NBUF], gsems[c % NBUF]
            )
        for c in range(nch):
            b = c % NBUF
            gathers[c].wait()
            e = c + NBUF - 1
            if e < nch and gathers[e] is None:
                be = e % NBUF
                if c >= 1:
                    scatters[c - 1].wait()
                gathers[e] = pltpu.async_copy(
                    lut_hbm.at[idx_v.at[e]], bufs[be], gsems[be]
                )
            scale_buf(bufs[b])
            scatters[c] = pltpu.async_copy(
                bufs[b], out_hbm.at[pl.ds(base + c * C, C)], ssems[b]
            )
        for c in range(nch):
            if c + NBUF >= nch:
                scatters[c].wait()

    return emb


@jax.jit
def kernel(X, lut):
    S0, S1 = X.shape
    V, D = lut.shape
    B = S0 * S1
    b_per_w = B // _NW
    C = 64
    NBUF = 3
    idx = X.reshape(_NW, b_per_w // C, C).astype(jnp.int32)
    emb = _make_emb_kernel(B, V, D, b_per_w, C, NBUF)
    out = emb(idx, lut)
    return out.reshape(S0, S1, D)
